# baseline (speedup 1.0000x reference)
import functools
import jax
import jax.numpy as jnp
from jax import lax
from jax.experimental import pallas as pl
from jax.experimental.pallas import tpu as pltpu
from jax.experimental.pallas import tpu_sc as plsc


_S = 512
_K = 64
_ROWS_PER_SLICE = 32
_N_ROWS = 160
_N_WORKERS = 32
_ROWS_PER_W = _N_ROWS // _N_WORKERS
_LANES = 16

_TC_ROWS = 96
_SC_DENSE_PER_W = (_N_ROWS - _TC_ROWS) // _N_WORKERS
_CH = 8
_CHUNKS = _S // _CH
_R = 8


def _sc_body(preds_hbm, xs_hbm, ys_hbm, out_hbm, sa_hbm,
             xs_v, ys_v, idx_v, g_v, buf0, buf1, d0, d1, sa_v, sem, sem2):
    wid = lax.axis_index("s") * 2 + lax.axis_index("c")

    pltpu.sync_copy(xs_hbm.at[wid], xs_v)
    pltpu.sync_copy(ys_hbm.at[wid], ys_v)

    row0 = wid * _ROWS_PER_W
    for r in range(_ROWS_PER_W):
        for c in range(_K // _LANES):
            o = r * _K + c * _LANES
            xv = xs_v[0, pl.ds(o, _LANES)]
            fv = xv + (row0 + r) * _S
            idx_v[r, pl.ds(c * _LANES, _LANES)] = fv

    bufs = (buf0, buf1)

    def start(r):
        return pltpu.async_copy(preds_hbm.at[idx_v.at[r]], bufs[r % 2], sem)

    def extract(r, cp):
        cp.wait()
        buf = bufs[r % 2]
        for c in range(_K // _LANES):
            rows = jax.lax.iota(jnp.int32, _LANES) + c * _LANES
            cols = ys_v[0, pl.ds(r * _K + c * _LANES, _LANES)]
            g_v[r, pl.ds(c * _LANES, _LANES)] = plsc.load_gather(
                buf, [rows, cols])

    cps = [None] * _ROWS_PER_W
    cps[0] = start(0)
    cps[1] = start(1)
    for r in range(_ROWS_PER_W):
        extract(r, cps[r])
        if r + 2 < _ROWS_PER_W:
            cps[r + 2] = start(r + 2)

    pltpu.sync_copy(g_v, out_hbm.at[wid])

    dbufs = (d0, d1)

    def chunk_src(big_row, ch):
        return preds_hbm.at[pl.ds(pl.multiple_of(big_row * _S + ch * _CH, _CH),
                                  _CH)]

    def process(dbuf, accs):
        new = list(accs)
        k = 0
        for r in range(_CH):
            for c in range(_S // _LANES):
                v = dbuf[r, pl.ds(c * _LANES, _LANES)]
                new[k % 8] = new[k % 8] + jnp.exp(v)
                k += 1
        return tuple(new)

    for t in range(_SC_DENSE_PER_W):
        big_row = _TC_ROWS + wid * _SC_DENSE_PER_W + t

        pltpu.async_copy(chunk_src(big_row, 0), d0, sem2)
        pltpu.async_copy(chunk_src(big_row, 1), d1, sem2)

        zero = jnp.zeros((_LANES,), jnp.float32)
        init = (zero,) * 8

        def body(i, accs, big_row=big_row):
            pltpu.make_async_copy(chunk_src(big_row, 0), d0, sem2).wait()
            accs = process(d0, accs)

            @pl.when(i < _CHUNKS // 2 - 1)
            def _():
                pltpu.async_copy(chunk_src(big_row, 2 * i + 2), d0, sem2)

            pltpu.make_async_copy(chunk_src(big_row, 1), d1, sem2).wait()
            accs = process(d1, accs)

            @pl.when(i < _CHUNKS // 2 - 1)
            def _():
                pltpu.async_copy(chunk_src(big_row, 2 * i + 3), d1, sem2)

            return accs

        accs = lax.fori_loop(0, _CHUNKS // 2, body, init, unroll=False)
        total = accs[0]
        for a in accs[1:]:
            total = total + a
        sa_v[t] = total

    pltpu.sync_copy(sa_v, sa_hbm.at[wid])


def _sc_call(preds_rows, xs_flat, ys_flat):
    mesh = plsc.VectorSubcoreMesh(core_axis_name="c", subcore_axis_name="s")
    return pl.kernel(
        _sc_body,
        mesh=mesh,
        compiler_params=pltpu.CompilerParams(needs_layout_passes=False),
        out_type=(
            jax.ShapeDtypeStruct((_N_WORKERS, _ROWS_PER_W, _K), jnp.float32),
            jax.ShapeDtypeStruct((_N_WORKERS, _SC_DENSE_PER_W, _LANES),
                                 jnp.float32),
        ),
        scratch_types=[
            pltpu.VMEM((1, _ROWS_PER_W * _K), jnp.int32),
            pltpu.VMEM((1, _ROWS_PER_W * _K), jnp.int32),
            pltpu.VMEM((_ROWS_PER_W, _K), jnp.int32),
            pltpu.VMEM((_ROWS_PER_W, _K), jnp.float32),
            pltpu.VMEM((_K, _S), jnp.float32),
            pltpu.VMEM((_K, _S), jnp.float32),
            pltpu.VMEM((_CH, _S), jnp.float32),
            pltpu.VMEM((_CH, _S), jnp.float32),
            pltpu.VMEM((_SC_DENSE_PER_W, _LANES), jnp.float32),
            pltpu.SemaphoreType.DMA,
            pltpu.SemaphoreType.DMA,
        ],
    )(preds_rows, xs_flat, ys_flat)


def _dense_kernel(p_ref, out_ref):
    s_all = jnp.sum(jnp.exp(p_ref[...]), axis=(1, 2))
    out_ref[...] = jnp.broadcast_to(s_all[None, :, None], (1, _R, 128))


def _combine_kernel(satc_ref, sasc_ref, g_ref, xr_ref, yr_ref, xc_ref, yc_ref,
                    out_ref):
    g = g_ref[...]
    x_r = xr_ref[...]
    y_r = yr_ref[...]
    x_c = xc_ref[...]
    y_c = yc_ref[...]

    flat_r = x_r * _S + y_r
    flat_c = x_c * _S + y_c
    eq = flat_c == flat_r
    ii = jax.lax.broadcasted_iota(jnp.int32, (_N_ROWS, _K, _K), 1)
    jj = jax.lax.broadcasted_iota(jnp.int32, (_N_ROWS, _K, _K), 2)
    dup = jnp.any(eq & (ii < jj), axis=1, keepdims=True)
    active = (x_r > 0) & (y_r > 0) & jnp.logical_not(dup)

    eg = jnp.exp(g)
    s_true_p = jnp.sum(jnp.where(active, eg, 0.0), axis=2)
    s_true_n = jnp.sum(jnp.where(active, 1.0 / eg, 0.0), axis=2)

    sa_tc = satc_ref[...].reshape(_TC_ROWS, 128)[:, 0:1]
    sa_sc = jnp.sum(sasc_ref[...], axis=2, keepdims=True).reshape(
        _N_ROWS - _TC_ROWS, 1)
    s_all = jnp.concatenate([sa_tc, sa_sc], axis=0)

    neg = jnp.log(1.0 + jnp.maximum(s_all - s_true_p, 0.0))
    pos = jnp.log(1.0 + s_true_n)
    row_loss = (neg + pos).reshape(_N_ROWS // _ROWS_PER_SLICE,
                                   _ROWS_PER_SLICE, 1)
    sl = jnp.sum(row_loss, axis=1) * (1.0 / _ROWS_PER_SLICE)
    out_ref[...] = jnp.broadcast_to(sl[None], (1, 5, 128))


@jax.jit
def kernel(y_preds, y_trues):
    p = y_preds.reshape(_N_ROWS, _S, _S)
    yt = y_trues.astype(jnp.int32).reshape(_N_ROWS, _K, 2)
    xs = yt[:, :, 0]
    ys = yt[:, :, 1]

    gathered, sa_sc = _sc_call(
        y_preds.reshape(_N_ROWS * _S, _S),
        xs.reshape(_N_WORKERS, 1, _ROWS_PER_W * _K),
        ys.reshape(_N_WORKERS, 1, _ROWS_PER_W * _K))

    sa_tc = pl.pallas_call(
        _dense_kernel,
        grid=(_TC_ROWS // _R,),
        in_specs=[pl.BlockSpec((_R, _S, _S), lambda i: (i, 0, 0))],
        out_specs=pl.BlockSpec((1, _R, 128), lambda i: (i, 0, 0)),
        out_shape=jax.ShapeDtypeStruct((_TC_ROWS // _R, _R, 128),
                                       jnp.float32),
    )(p)

    x_r = xs.reshape(_N_ROWS, 1, _K)
    y_r = ys.reshape(_N_ROWS, 1, _K)
    x_c = xs.reshape(_N_ROWS, _K, 1)
    y_c = ys.reshape(_N_ROWS, _K, 1)
    g = gathered.reshape(_N_ROWS, 1, _K)

    out = pl.pallas_call(
        _combine_kernel,
        grid=(1,),
        in_specs=[
            pl.BlockSpec((_TC_ROWS // _R, _R, 128), lambda i: (0, 0, 0)),
            pl.BlockSpec((_N_WORKERS, _SC_DENSE_PER_W, _LANES),
                         lambda i: (0, 0, 0)),
            pl.BlockSpec((_N_ROWS, 1, _K), lambda i: (0, 0, 0)),
            pl.BlockSpec((_N_ROWS, 1, _K), lambda i: (0, 0, 0)),
            pl.BlockSpec((_N_ROWS, 1, _K), lambda i: (0, 0, 0)),
            pl.BlockSpec((_N_ROWS, _K, 1), lambda i: (0, 0, 0)),
            pl.BlockSpec((_N_ROWS, _K, 1), lambda i: (0, 0, 0)),
        ],
        out_specs=pl.BlockSpec((1, 5, 128), lambda i: (0, 0, 0)),
        out_shape=jax.ShapeDtypeStruct((1, 5, 128), jnp.float32),
    )(sa_tc, sa_sc, g, x_r, y_r, x_c, y_c)

    losses = out[0, :, 0]
    loss = jnp.mean(losses)
    return (loss, losses[0], losses[1], losses[2], losses[3], losses[4])


# speedup vs baseline: 1.2880x; 1.2880x over previous
import functools
import jax
import jax.numpy as jnp
from jax import lax
from jax.experimental import pallas as pl
from jax.experimental.pallas import tpu as pltpu
from jax.experimental.pallas import tpu_sc as plsc


_S = 512
_K = 64
_ROWS_PER_SLICE = 32
_N_ROWS = 160
_N_WORKERS = 32
_ROWS_PER_W = _N_ROWS // _N_WORKERS
_LANES = 16

_TC_ROWS = 128
_SC_DENSE_PER_W = (_N_ROWS - _TC_ROWS) // _N_WORKERS
_CH = 8
_CHUNKS = _S // _CH
_R = 8


def _sc_body(preds_hbm, xs_hbm, ys_hbm, out_hbm, sa_hbm,
             xs_v, ys_v, idx_v, g_v, buf0, buf1, d0, d1, sa_v, sem, sem2):
    wid = lax.axis_index("s") * 2 + lax.axis_index("c")

    pltpu.sync_copy(xs_hbm.at[wid], xs_v)
    pltpu.sync_copy(ys_hbm.at[wid], ys_v)

    row0 = wid * _ROWS_PER_W
    for r in range(_ROWS_PER_W):
        for c in range(_K // _LANES):
            o = r * _K + c * _LANES
            xv = xs_v[0, pl.ds(o, _LANES)]
            fv = xv + (row0 + r) * _S
            idx_v[r, pl.ds(c * _LANES, _LANES)] = fv

    bufs = (buf0, buf1)

    def start(r):
        return pltpu.async_copy(preds_hbm.at[idx_v.at[r]], bufs[r % 2], sem)

    def extract(r, cp):
        cp.wait()
        buf = bufs[r % 2]
        for c in range(_K // _LANES):
            rows = jax.lax.iota(jnp.int32, _LANES) + c * _LANES
            cols = ys_v[0, pl.ds(r * _K + c * _LANES, _LANES)]
            g_v[r, pl.ds(c * _LANES, _LANES)] = plsc.load_gather(
                buf, [rows, cols])

    cps = [None] * _ROWS_PER_W
    cps[0] = start(0)
    cps[1] = start(1)
    for r in range(_ROWS_PER_W):
        extract(r, cps[r])
        if r + 2 < _ROWS_PER_W:
            cps[r + 2] = start(r + 2)

    pltpu.sync_copy(g_v, out_hbm.at[wid])

    dbufs = (d0, d1)

    def chunk_src(big_row, ch):
        return preds_hbm.at[pl.ds(pl.multiple_of(big_row * _S + ch * _CH, _CH),
                                  _CH)]

    def process(dbuf, accs):
        new = list(accs)
        k = 0
        for r in range(_CH):
            for c in range(_S // _LANES):
                v = dbuf[r, pl.ds(c * _LANES, _LANES)]
                new[k % 8] = new[k % 8] + jnp.exp(v)
                k += 1
        return tuple(new)

    for t in range(_SC_DENSE_PER_W):
        big_row = _TC_ROWS + wid * _SC_DENSE_PER_W + t

        pltpu.async_copy(chunk_src(big_row, 0), d0, sem2)
        pltpu.async_copy(chunk_src(big_row, 1), d1, sem2)

        zero = jnp.zeros((_LANES,), jnp.float32)
        init = (zero,) * 8

        def body(i, accs, big_row=big_row):
            pltpu.make_async_copy(chunk_src(big_row, 0), d0, sem2).wait()
            accs = process(d0, accs)

            @pl.when(i < _CHUNKS // 2 - 1)
            def _():
                pltpu.async_copy(chunk_src(big_row, 2 * i + 2), d0, sem2)

            pltpu.make_async_copy(chunk_src(big_row, 1), d1, sem2).wait()
            accs = process(d1, accs)

            @pl.when(i < _CHUNKS // 2 - 1)
            def _():
                pltpu.async_copy(chunk_src(big_row, 2 * i + 3), d1, sem2)

            return accs

        accs = lax.fori_loop(0, _CHUNKS // 2, body, init, unroll=False)
        total = accs[0]
        for a in accs[1:]:
            total = total + a
        sa_v[t] = total

    pltpu.sync_copy(sa_v, sa_hbm.at[wid])


def _sc_call(preds_rows, xs_flat, ys_flat):
    mesh = plsc.VectorSubcoreMesh(core_axis_name="c", subcore_axis_name="s")
    return pl.kernel(
        _sc_body,
        mesh=mesh,
        compiler_params=pltpu.CompilerParams(needs_layout_passes=False),
        out_type=(
            jax.ShapeDtypeStruct((_N_WORKERS, _ROWS_PER_W, _K), jnp.float32),
            jax.ShapeDtypeStruct((_N_WORKERS, _SC_DENSE_PER_W, _LANES),
                                 jnp.float32),
        ),
        scratch_types=[
            pltpu.VMEM((1, _ROWS_PER_W * _K), jnp.int32),
            pltpu.VMEM((1, _ROWS_PER_W * _K), jnp.int32),
            pltpu.VMEM((_ROWS_PER_W, _K), jnp.int32),
            pltpu.VMEM((_ROWS_PER_W, _K), jnp.float32),
            pltpu.VMEM((_K, _S), jnp.float32),
            pltpu.VMEM((_K, _S), jnp.float32),
            pltpu.VMEM((_CH, _S), jnp.float32),
            pltpu.VMEM((_CH, _S), jnp.float32),
            pltpu.VMEM((_SC_DENSE_PER_W, _LANES), jnp.float32),
            pltpu.SemaphoreType.DMA,
            pltpu.SemaphoreType.DMA,
        ],
    )(preds_rows, xs_flat, ys_flat)


def _dense_kernel(p_ref, out_ref):
    s_all = jnp.sum(jnp.exp(p_ref[...]), axis=(1, 2))
    out_ref[...] = jnp.broadcast_to(s_all[None, :, None], (1, _R, 128))


def _combine_kernel(satc_ref, sasc_ref, g_ref, xr_ref, yr_ref, xc_ref, yc_ref,
                    out_ref):
    g = g_ref[...]
    x_r = xr_ref[...]
    y_r = yr_ref[...]
    x_c = xc_ref[...]
    y_c = yc_ref[...]

    flat_r = x_r * _S + y_r
    flat_c = x_c * _S + y_c
    eq = flat_c == flat_r
    ii = jax.lax.broadcasted_iota(jnp.int32, (_N_ROWS, _K, _K), 1)
    jj = jax.lax.broadcasted_iota(jnp.int32, (_N_ROWS, _K, _K), 2)
    dup = jnp.any(eq & (ii < jj), axis=1, keepdims=True)
    active = (x_r > 0) & (y_r > 0) & jnp.logical_not(dup)

    eg = jnp.exp(g)
    s_true_p = jnp.sum(jnp.where(active, eg, 0.0), axis=2)
    s_true_n = jnp.sum(jnp.where(active, 1.0 / eg, 0.0), axis=2)

    sa_tc = satc_ref[...].reshape(_TC_ROWS, 128)[:, 0:1]
    sa_sc = jnp.sum(sasc_ref[...], axis=2, keepdims=True).reshape(
        _N_ROWS - _TC_ROWS, 1)
    s_all = jnp.concatenate([sa_tc, sa_sc], axis=0)

    neg = jnp.log(1.0 + jnp.maximum(s_all - s_true_p, 0.0))
    pos = jnp.log(1.0 + s_true_n)
    row_loss = (neg + pos).reshape(_N_ROWS // _ROWS_PER_SLICE,
                                   _ROWS_PER_SLICE, 1)
    sl = jnp.sum(row_loss, axis=1) * (1.0 / _ROWS_PER_SLICE)
    out_ref[...] = jnp.broadcast_to(sl[None], (1, 5, 128))


@jax.jit
def kernel(y_preds, y_trues):
    p = y_preds.reshape(_N_ROWS, _S, _S)
    yt = y_trues.astype(jnp.int32).reshape(_N_ROWS, _K, 2)
    xs = yt[:, :, 0]
    ys = yt[:, :, 1]

    gathered, sa_sc = _sc_call(
        y_preds.reshape(_N_ROWS * _S, _S),
        xs.reshape(_N_WORKERS, 1, _ROWS_PER_W * _K),
        ys.reshape(_N_WORKERS, 1, _ROWS_PER_W * _K))

    sa_tc = pl.pallas_call(
        _dense_kernel,
        grid=(_TC_ROWS // _R,),
        in_specs=[pl.BlockSpec((_R, _S, _S), lambda i: (i, 0, 0))],
        out_specs=pl.BlockSpec((1, _R, 128), lambda i: (i, 0, 0)),
        out_shape=jax.ShapeDtypeStruct((_TC_ROWS // _R, _R, 128),
                                       jnp.float32),
    )(p)

    x_r = xs.reshape(_N_ROWS, 1, _K)
    y_r = ys.reshape(_N_ROWS, 1, _K)
    x_c = xs.reshape(_N_ROWS, _K, 1)
    y_c = ys.reshape(_N_ROWS, _K, 1)
    g = gathered.reshape(_N_ROWS, 1, _K)

    out = pl.pallas_call(
        _combine_kernel,
        grid=(1,),
        in_specs=[
            pl.BlockSpec((_TC_ROWS // _R, _R, 128), lambda i: (0, 0, 0)),
            pl.BlockSpec((_N_WORKERS, _SC_DENSE_PER_W, _LANES),
                         lambda i: (0, 0, 0)),
            pl.BlockSpec((_N_ROWS, 1, _K), lambda i: (0, 0, 0)),
            pl.BlockSpec((_N_ROWS, 1, _K), lambda i: (0, 0, 0)),
            pl.BlockSpec((_N_ROWS, 1, _K), lambda i: (0, 0, 0)),
            pl.BlockSpec((_N_ROWS, _K, 1), lambda i: (0, 0, 0)),
            pl.BlockSpec((_N_ROWS, _K, 1), lambda i: (0, 0, 0)),
        ],
        out_specs=pl.BlockSpec((1, 5, 128), lambda i: (0, 0, 0)),
        out_shape=jax.ShapeDtypeStruct((1, 5, 128), jnp.float32),
    )(sa_tc, sa_sc, g, x_r, y_r, x_c, y_c)

    losses = out[0, :, 0]
    loss = jnp.mean(losses)
    return (loss, losses[0], losses[1], losses[2], losses[3], losses[4])


# speedup vs baseline: 1.3039x; 1.0124x over previous
import functools
import jax
import jax.numpy as jnp
from jax import lax
from jax.experimental import pallas as pl
from jax.experimental.pallas import tpu as pltpu
from jax.experimental.pallas import tpu_sc as plsc


_S = 512
_K = 64
_ROWS_PER_SLICE = 32
_N_ROWS = 160
_N_WORKERS = 32
_ROWS_PER_W = _N_ROWS // _N_WORKERS
_LANES = 16

_R = 8


def _sc_body(preds_hbm, xs_hbm, ys_hbm, out_hbm,
             xs_v, ys_v, idx_v, g_v, buf0, buf1, sem):
    wid = lax.axis_index("s") * 2 + lax.axis_index("c")

    pltpu.sync_copy(xs_hbm.at[wid], xs_v)
    pltpu.sync_copy(ys_hbm.at[wid], ys_v)

    row0 = wid * _ROWS_PER_W
    for r in range(_ROWS_PER_W):
        for c in range(_K // _LANES):
            o = r * _K + c * _LANES
            xv = xs_v[0, pl.ds(o, _LANES)]
            fv = xv + (row0 + r) * _S
            idx_v[r, pl.ds(c * _LANES, _LANES)] = fv

    bufs = (buf0, buf1)

    def start(r):
        return pltpu.async_copy(preds_hbm.at[idx_v.at[r]], bufs[r % 2], sem)

    def extract(r, cp):
        cp.wait()
        buf = bufs[r % 2]
        for c in range(_K // _LANES):
            rows = jax.lax.iota(jnp.int32, _LANES) + c * _LANES
            cols = ys_v[0, pl.ds(r * _K + c * _LANES, _LANES)]
            g_v[r, pl.ds(c * _LANES, _LANES)] = plsc.load_gather(
                buf, [rows, cols])

    cps = [None] * _ROWS_PER_W
    cps[0] = start(0)
    cps[1] = start(1)
    for r in range(_ROWS_PER_W):
        extract(r, cps[r])
        if r + 2 < _ROWS_PER_W:
            cps[r + 2] = start(r + 2)

    pltpu.sync_copy(g_v, out_hbm.at[wid])


def _sc_call(preds_rows, xs_flat, ys_flat):
    mesh = plsc.VectorSubcoreMesh(core_axis_name="c", subcore_axis_name="s")
    return pl.kernel(
        _sc_body,
        mesh=mesh,
        compiler_params=pltpu.CompilerParams(needs_layout_passes=False),
        out_type=jax.ShapeDtypeStruct((_N_WORKERS, _ROWS_PER_W, _K),
                                      jnp.float32),
        scratch_types=[
            pltpu.VMEM((1, _ROWS_PER_W * _K), jnp.int32),
            pltpu.VMEM((1, _ROWS_PER_W * _K), jnp.int32),
            pltpu.VMEM((_ROWS_PER_W, _K), jnp.int32),
            pltpu.VMEM((_ROWS_PER_W, _K), jnp.float32),
            pltpu.VMEM((_K, _S), jnp.float32),
            pltpu.VMEM((_K, _S), jnp.float32),
            pltpu.SemaphoreType.DMA,
        ],
    )(preds_rows, xs_flat, ys_flat)


def _dense_kernel(p_ref, out_ref):
    s_all = jnp.sum(jnp.exp(p_ref[...]), axis=(1, 2))
    out_ref[...] = jnp.broadcast_to(s_all[None, :, None], (1, _R, 128))


def _combine_kernel(satc_ref, g_ref, xr_ref, yr_ref, xc_ref, yc_ref,
                    out_ref):
    g = g_ref[...]
    x_r = xr_ref[...]
    y_r = yr_ref[...]
    x_c = xc_ref[...]
    y_c = yc_ref[...]

    flat_r = x_r * _S + y_r
    flat_c = x_c * _S + y_c
    eq = flat_c == flat_r
    ii = jax.lax.broadcasted_iota(jnp.int32, (_N_ROWS, _K, _K), 1)
    jj = jax.lax.broadcasted_iota(jnp.int32, (_N_ROWS, _K, _K), 2)
    dup = jnp.any(eq & (ii < jj), axis=1, keepdims=True)
    active = (x_r > 0) & (y_r > 0) & jnp.logical_not(dup)

    eg = jnp.exp(g)
    s_true_p = jnp.sum(jnp.where(active, eg, 0.0), axis=2)
    s_true_n = jnp.sum(jnp.where(active, 1.0 / eg, 0.0), axis=2)

    s_all = satc_ref[...].reshape(_N_ROWS, 128)[:, 0:1]

    neg = jnp.log(1.0 + jnp.maximum(s_all - s_true_p, 0.0))
    pos = jnp.log(1.0 + s_true_n)
    row_loss = (neg + pos).reshape(_N_ROWS // _ROWS_PER_SLICE,
                                   _ROWS_PER_SLICE, 1)
    sl = jnp.sum(row_loss, axis=1) * (1.0 / _ROWS_PER_SLICE)
    out_ref[...] = jnp.broadcast_to(sl[None], (1, 5, 128))


@jax.jit
def kernel(y_preds, y_trues):
    p = y_preds.reshape(_N_ROWS, _S, _S)
    yt = y_trues.astype(jnp.int32).reshape(_N_ROWS, _K, 2)
    xs = yt[:, :, 0]
    ys = yt[:, :, 1]

    gathered = _sc_call(
        y_preds.reshape(_N_ROWS * _S, _S),
        xs.reshape(_N_WORKERS, 1, _ROWS_PER_W * _K),
        ys.reshape(_N_WORKERS, 1, _ROWS_PER_W * _K))

    sa_tc = pl.pallas_call(
        _dense_kernel,
        grid=(_N_ROWS // _R,),
        in_specs=[pl.BlockSpec((_R, _S, _S), lambda i: (i, 0, 0))],
        out_specs=pl.BlockSpec((1, _R, 128), lambda i: (i, 0, 0)),
        out_shape=jax.ShapeDtypeStruct((_N_ROWS // _R, _R, 128),
                                       jnp.float32),
    )(p)

    x_r = xs.reshape(_N_ROWS, 1, _K)
    y_r = ys.reshape(_N_ROWS, 1, _K)
    x_c = xs.reshape(_N_ROWS, _K, 1)
    y_c = ys.reshape(_N_ROWS, _K, 1)
    g = gathered.reshape(_N_ROWS, 1, _K)

    out = pl.pallas_call(
        _combine_kernel,
        grid=(1,),
        in_specs=[
            pl.BlockSpec((_N_ROWS // _R, _R, 128), lambda i: (0, 0, 0)),
            pl.BlockSpec((_N_ROWS, 1, _K), lambda i: (0, 0, 0)),
            pl.BlockSpec((_N_ROWS, 1, _K), lambda i: (0, 0, 0)),
            pl.BlockSpec((_N_ROWS, 1, _K), lambda i: (0, 0, 0)),
            pl.BlockSpec((_N_ROWS, _K, 1), lambda i: (0, 0, 0)),
            pl.BlockSpec((_N_ROWS, _K, 1), lambda i: (0, 0, 0)),
        ],
        out_specs=pl.BlockSpec((1, 5, 128), lambda i: (0, 0, 0)),
        out_shape=jax.ShapeDtypeStruct((1, 5, 128), jnp.float32),
    )(sa_tc, g, x_r, y_r, x_c, y_c)

    losses = out[0, :, 0]
    loss = jnp.mean(losses)
    return (loss, losses[0], losses[1], losses[2], losses[3], losses[4])


# speedup vs baseline: 1.3654x; 1.0472x over previous
import jax
import jax.numpy as jnp
from jax import lax
from jax.experimental import pallas as pl
from jax.experimental.pallas import tpu as pltpu
from jax.experimental.pallas import tpu_sc as plsc


_S = 512
_K = 64
_ROWS_PER_SLICE = 32
_N_ROWS = 160
_ROW_ELEMS = _S * _S
_N_WORKERS = 32
_ROWS_PER_W = _N_ROWS // _N_WORKERS
_LANES = 16


def _sc_gather_body(preds_hbm, xs_hbm, ys_hbm, out_hbm,
                    xs_v, ys_v, idx_v, g_v, buf0, buf1, sem):
    wid = lax.axis_index("s") * 2 + lax.axis_index("c")

    pltpu.sync_copy(xs_hbm.at[wid], xs_v)
    pltpu.sync_copy(ys_hbm.at[wid], ys_v)

    row0 = wid * _ROWS_PER_W
    for r in range(_ROWS_PER_W):
        for c in range(_K // _LANES):
            o = r * _K + c * _LANES
            xv = xs_v[0, pl.ds(o, _LANES)]
            fv = xv + (row0 + r) * _S
            idx_v[r, pl.ds(c * _LANES, _LANES)] = fv

    bufs = (buf0, buf1)

    def start(r):
        return pltpu.async_copy(preds_hbm.at[idx_v.at[r]], bufs[r % 2], sem)

    def extract(r, cp):
        cp.wait()
        buf = bufs[r % 2]
        for c in range(_K // _LANES):
            rows = jax.lax.iota(jnp.int32, _LANES) + c * _LANES
            cols = ys_v[0, pl.ds(r * _K + c * _LANES, _LANES)]
            g_v[r, pl.ds(c * _LANES, _LANES)] = plsc.load_gather(
                buf, [rows, cols])

    cps = [None] * _ROWS_PER_W
    cps[0] = start(0)
    cps[1] = start(1)
    for r in range(_ROWS_PER_W):
        extract(r, cps[r])
        if r + 2 < _ROWS_PER_W:
            cps[r + 2] = start(r + 2)

    pltpu.sync_copy(g_v, out_hbm.at[wid])


def _sc_gather(preds_rows, xs_flat, ys_flat):
    mesh = plsc.VectorSubcoreMesh(core_axis_name="c", subcore_axis_name="s")
    return pl.kernel(
        _sc_gather_body,
        mesh=mesh,
        compiler_params=pltpu.CompilerParams(needs_layout_passes=False),
        out_type=jax.ShapeDtypeStruct((_N_WORKERS, _ROWS_PER_W, _K),
                                      jnp.float32),
        scratch_types=[
            pltpu.VMEM((1, _ROWS_PER_W * _K), jnp.int32),
            pltpu.VMEM((1, _ROWS_PER_W * _K), jnp.int32),
            pltpu.VMEM((_ROWS_PER_W, _K), jnp.int32),
            pltpu.VMEM((_ROWS_PER_W, _K), jnp.float32),
            pltpu.VMEM((_K, _S), jnp.float32),
            pltpu.VMEM((_K, _S), jnp.float32),
            pltpu.SemaphoreType.DMA,
        ],
    )(preds_rows, xs_flat, ys_flat)


_R = 8


def _row_loss_kernel(p_ref, g_ref, xr_ref, yr_ref, xc_ref, yc_ref, out_ref):
    i = pl.program_id(0)

    @pl.when(i % (_ROWS_PER_SLICE // _R) == 0)
    def _init():
        out_ref[...] = jnp.zeros_like(out_ref)

    p = p_ref[...]
    g = g_ref[...]
    x_r = xr_ref[...]
    y_r = yr_ref[...]
    x_c = xc_ref[...]
    y_c = yc_ref[...]

    flat_r = x_r * _S + y_r
    flat_c = x_c * _S + y_c
    eq = flat_c == flat_r
    ii = jax.lax.broadcasted_iota(jnp.int32, (_R, _K, _K), 1)
    jj = jax.lax.broadcasted_iota(jnp.int32, (_R, _K, _K), 2)
    dup = jnp.any(eq & (ii < jj), axis=1, keepdims=True)
    active = (x_r > 0) & (y_r > 0) & jnp.logical_not(dup)

    eg = jnp.exp(g)
    s_true_p = jnp.sum(jnp.where(active, eg, 0.0), axis=(1, 2))
    s_true_n = jnp.sum(jnp.where(active, 1.0 / eg, 0.0), axis=(1, 2))

    s_all = jnp.sum(jnp.exp(p), axis=(1, 2))

    neg = jnp.log(1.0 + jnp.maximum(s_all - s_true_p, 0.0))
    pos = jnp.log(1.0 + s_true_n)
    loss = jnp.sum(neg + pos) * (1.0 / _ROWS_PER_SLICE)

    out_ref[...] += jnp.full((1, 1, 128), loss, jnp.float32)


@jax.jit
def kernel(y_preds, y_trues):
    p = y_preds.reshape(_N_ROWS, _S, _S)
    yt = y_trues.astype(jnp.int32).reshape(_N_ROWS, _K, 2)
    xs = yt[:, :, 0]
    ys = yt[:, :, 1]

    gathered = _sc_gather(
        y_preds.reshape(_N_ROWS * _S, _S),
        xs.reshape(_N_WORKERS, 1, _ROWS_PER_W * _K),
        ys.reshape(_N_WORKERS, 1, _ROWS_PER_W * _K))

    x_r = xs.reshape(_N_ROWS, 1, _K)
    y_r = ys.reshape(_N_ROWS, 1, _K)
    x_c = xs.reshape(_N_ROWS, _K, 1)
    y_c = ys.reshape(_N_ROWS, _K, 1)
    g = gathered.reshape(_N_ROWS, 1, _K)

    out = pl.pallas_call(
        _row_loss_kernel,
        grid=(_N_ROWS // _R,),
        in_specs=[
            pl.BlockSpec((_R, _S, _S), lambda i: (i, 0, 0)),
            pl.BlockSpec((_R, 1, _K), lambda i: (i, 0, 0)),
            pl.BlockSpec((_R, 1, _K), lambda i: (i, 0, 0)),
            pl.BlockSpec((_R, 1, _K), lambda i: (i, 0, 0)),
            pl.BlockSpec((_R, _K, 1), lambda i: (i, 0, 0)),
            pl.BlockSpec((_R, _K, 1), lambda i: (i, 0, 0)),
        ],
        out_specs=pl.BlockSpec((1, 1, 128),
                               lambda i: (i // (_ROWS_PER_SLICE // _R), 0, 0)),
        out_shape=jax.ShapeDtypeStruct((_N_ROWS // _ROWS_PER_SLICE, 1, 128),
                                       jnp.float32),
    )(p, g, x_r, y_r, x_c, y_c)

    losses = out[:, 0, 0]
    loss = jnp.mean(losses)
    return (loss, losses[0], losses[1], losses[2], losses[3], losses[4])


# speedup vs baseline: 1.3720x; 1.0048x over previous
import jax
import jax.numpy as jnp
from jax import lax
from jax.experimental import pallas as pl
from jax.experimental.pallas import tpu as pltpu
from jax.experimental.pallas import tpu_sc as plsc


_S = 512
_K = 64
_ROWS_PER_SLICE = 32
_N_ROWS = 160
_ROW_ELEMS = _S * _S
_N_WORKERS = 32
_ROWS_PER_W = _N_ROWS // _N_WORKERS
_LANES = 16


def _sc_gather_body(preds_hbm, xs_hbm, ys_hbm, out_hbm,
                    xs_v, ys_v, idx_v, g_v, buf0, buf1, sem):
    wid = lax.axis_index("s") * 2 + lax.axis_index("c")

    pltpu.sync_copy(xs_hbm.at[wid], xs_v)
    pltpu.sync_copy(ys_hbm.at[wid], ys_v)

    row0 = wid * _ROWS_PER_W
    for r in range(_ROWS_PER_W):
        for c in range(_K // _LANES):
            o = r * _K + c * _LANES
            xv = xs_v[0, pl.ds(o, _LANES)]
            fv = xv + (row0 + r) * _S
            idx_v[r, pl.ds(c * _LANES, _LANES)] = fv

    bufs = (buf0, buf1)

    def start(r):
        return pltpu.async_copy(preds_hbm.at[idx_v.at[r]], bufs[r % 2], sem)

    def extract(r, cp):
        cp.wait()
        buf = bufs[r % 2]
        for c in range(_K // _LANES):
            rows = jax.lax.iota(jnp.int32, _LANES) + c * _LANES
            cols = ys_v[0, pl.ds(r * _K + c * _LANES, _LANES)]
            g_v[r, pl.ds(c * _LANES, _LANES)] = plsc.load_gather(
                buf, [rows, cols])

    cps = [None] * _ROWS_PER_W
    cps[0] = start(0)
    cps[1] = start(1)
    for r in range(_ROWS_PER_W):
        extract(r, cps[r])
        if r + 2 < _ROWS_PER_W:
            cps[r + 2] = start(r + 2)

    pltpu.sync_copy(g_v, out_hbm.at[wid])


def _sc_gather(preds_rows, xs_flat, ys_flat):
    mesh = plsc.VectorSubcoreMesh(core_axis_name="c", subcore_axis_name="s")
    return pl.kernel(
        _sc_gather_body,
        mesh=mesh,
        compiler_params=pltpu.CompilerParams(needs_layout_passes=False),
        out_type=jax.ShapeDtypeStruct((_N_WORKERS, _ROWS_PER_W, _K),
                                      jnp.float32),
        scratch_types=[
            pltpu.VMEM((1, _ROWS_PER_W * _K), jnp.int32),
            pltpu.VMEM((1, _ROWS_PER_W * _K), jnp.int32),
            pltpu.VMEM((_ROWS_PER_W, _K), jnp.int32),
            pltpu.VMEM((_ROWS_PER_W, _K), jnp.float32),
            pltpu.VMEM((_K, _S), jnp.float32),
            pltpu.VMEM((_K, _S), jnp.float32),
            pltpu.SemaphoreType.DMA,
        ],
    )(preds_rows, xs_flat, ys_flat)


_R = 8


def _row_loss_kernel(pa_ref, pb_ref, g_ref, xr_ref, yr_ref, xc_ref, yc_ref,
                     out_ref):
    i = pl.program_id(0)

    @pl.when(i % (_ROWS_PER_SLICE // _R) == 0)
    def _init():
        out_ref[...] = jnp.zeros_like(out_ref)

    g = g_ref[...]
    x_r = xr_ref[...]
    y_r = yr_ref[...]
    x_c = xc_ref[...]
    y_c = yc_ref[...]

    flat_r = x_r * _S + y_r
    flat_c = x_c * _S + y_c
    eq = flat_c == flat_r
    ii = jax.lax.broadcasted_iota(jnp.int32, (_R, _K, _K), 1)
    jj = jax.lax.broadcasted_iota(jnp.int32, (_R, _K, _K), 2)
    dup = jnp.any(eq & (ii < jj), axis=1, keepdims=True)
    active = (x_r > 0) & (y_r > 0) & jnp.logical_not(dup)

    eg = jnp.exp(g)
    s_true_p = jnp.sum(jnp.where(active, eg, 0.0), axis=(1, 2))
    s_true_n = jnp.sum(jnp.where(active, 1.0 / eg, 0.0), axis=(1, 2))

    s_all = (jnp.sum(jnp.exp(pa_ref[...]), axis=(1, 2))
             + jnp.sum(jnp.exp(pb_ref[...]), axis=(1, 2)))

    neg = jnp.log(1.0 + jnp.maximum(s_all - s_true_p, 0.0))
    pos = jnp.log(1.0 + s_true_n)
    loss = jnp.sum(neg + pos) * (1.0 / _ROWS_PER_SLICE)

    out_ref[...] += jnp.full((1, 1, 128), loss, jnp.float32)


@jax.jit
def kernel(y_preds, y_trues):
    p = y_preds.reshape(_N_ROWS, _S, _S)
    yt = y_trues.astype(jnp.int32).reshape(_N_ROWS, _K, 2)
    xs = yt[:, :, 0]
    ys = yt[:, :, 1]

    gathered = _sc_gather(
        y_preds.reshape(_N_ROWS * _S, _S),
        xs.reshape(_N_WORKERS, 1, _ROWS_PER_W * _K),
        ys.reshape(_N_WORKERS, 1, _ROWS_PER_W * _K))

    x_r = xs.reshape(_N_ROWS, 1, _K)
    y_r = ys.reshape(_N_ROWS, 1, _K)
    x_c = xs.reshape(_N_ROWS, _K, 1)
    y_c = ys.reshape(_N_ROWS, _K, 1)
    g = gathered.reshape(_N_ROWS, 1, _K)

    out = pl.pallas_call(
        _row_loss_kernel,
        grid=(_N_ROWS // _R,),
        in_specs=[
            pl.BlockSpec((_R, _S // 2, _S), lambda i: (i, 0, 0)),
            pl.BlockSpec((_R, _S // 2, _S), lambda i: (i, 1, 0)),
            pl.BlockSpec((_R, 1, _K), lambda i: (i, 0, 0)),
            pl.BlockSpec((_R, 1, _K), lambda i: (i, 0, 0)),
            pl.BlockSpec((_R, 1, _K), lambda i: (i, 0, 0)),
            pl.BlockSpec((_R, _K, 1), lambda i: (i, 0, 0)),
            pl.BlockSpec((_R, _K, 1), lambda i: (i, 0, 0)),
        ],
        out_specs=pl.BlockSpec((1, 1, 128),
                               lambda i: (i // (_ROWS_PER_SLICE // _R), 0, 0)),
        out_shape=jax.ShapeDtypeStruct((_N_ROWS // _ROWS_PER_SLICE, 1, 128),
                                       jnp.float32),
    )(p, p, g, x_r, y_r, x_c, y_c)

    losses = out[:, 0, 0]
    loss = jnp.mean(losses)
    return (loss, losses[0], losses[1], losses[2], losses[3], losses[4])
